# speedup vs baseline: 1.0434x; 1.0434x over previous
import math

import jax
import jax.numpy as jnp
from jax import lax
from jax.experimental import pallas as pl
from jax.experimental.pallas import tpu as pltpu
from jax.experimental.pallas import tpu_sc as plsc

VOCAB = 100000
D_MODEL = 1024
SCALE = math.sqrt(D_MODEL)

NC = 2
NS = 16
NW = NC * NS
LANES = 16

B_TOTAL = 4 * 4096
B_PER_W = B_TOTAL // NW
CH = 64
N_CH = B_PER_W // CH


def _emb_kernel(idx_hbm, table_hbm, out_hbm, idx_v, rows_v, sem):
    wid = lax.axis_index("s") * NC + lax.axis_index("c")
    base = wid * B_PER_W
    pltpu.sync_copy(idx_hbm.at[pl.ds(base, B_PER_W)], idx_v)

    def chunk_body(ci, _):
        off = ci * CH
        pltpu.async_copy(
            table_hbm.at[idx_v.at[pl.ds(off, CH)]], rows_v, sem
        ).wait()

        def scale_row(r, _):
            for j in range(D_MODEL // LANES):
                col = j * LANES
                rows_v[r, pl.ds(col, LANES)] = (
                    rows_v[r, pl.ds(col, LANES)] * SCALE
                )
            return 0

        lax.fori_loop(0, CH, scale_row, 0)
        pltpu.sync_copy(rows_v, out_hbm.at[pl.ds(base + off, CH)])
        return 0

    lax.fori_loop(0, N_CH, chunk_body, 0)


@jax.jit
def kernel(input, table):
    idx = input.reshape(-1).astype(jnp.int32)
    mesh = plsc.VectorSubcoreMesh(core_axis_name="c", subcore_axis_name="s")
    out = pl.kernel(
        _emb_kernel,
        out_type=jax.ShapeDtypeStruct((B_TOTAL, D_MODEL), jnp.float32),
        mesh=mesh,
        scratch_types=[
            pltpu.VMEM((B_PER_W,), jnp.int32),
            pltpu.VMEM((CH, D_MODEL), jnp.float32),
            pltpu.SemaphoreType.DMA,
        ],
    )(idx, table)
    return out.reshape(input.shape + (D_MODEL,))


# speedup vs baseline: 1.3293x; 1.2740x over previous
import math

import jax
import jax.numpy as jnp
from jax import lax
from jax.experimental import pallas as pl
from jax.experimental.pallas import tpu as pltpu
from jax.experimental.pallas import tpu_sc as plsc

VOCAB = 100000
D_MODEL = 1024
SCALE = math.sqrt(D_MODEL)

NC = 2
NS = 16
NW = NC * NS
LANES = 16

B_TOTAL = 4 * 4096
B_PER_W = B_TOTAL // NW
CH = 32
N_CH = B_PER_W // CH
NBUF = 2
N_ROUNDS = N_CH // NBUF


def _emb_kernel(idx_hbm, table_hbm, out_hbm, idx_v, rows0, rows1,
                gsem0, gsem1, ssem0, ssem1):
    wid = lax.axis_index("s") * NC + lax.axis_index("c")
    base = wid * B_PER_W
    rows = (rows0, rows1)
    gsem = (gsem0, gsem1)
    ssem = (ssem0, ssem1)

    pltpu.sync_copy(idx_hbm.at[pl.ds(base, B_PER_W)], idx_v)

    def gather_desc(b, ci):
        return pltpu.make_async_copy(
            table_hbm.at[idx_v.at[pl.ds(ci * CH, CH)]], rows[b], gsem[b]
        )

    def scatter_desc(b, ci):
        return pltpu.make_async_copy(
            rows[b], out_hbm.at[pl.ds(base + ci * CH, CH)], ssem[b]
        )

    def scale_buf(b):
        def scale_row(r, _):
            for j in range(D_MODEL // LANES):
                col = j * LANES
                rows[b][r, pl.ds(col, LANES)] = (
                    rows[b][r, pl.ds(col, LANES)] * SCALE
                )
            return 0

        lax.fori_loop(0, CH, scale_row, 0)

    for b in range(NBUF):
        gather_desc(b, b).start()

    def round_body(r, _):
        for b in range(NBUF):
            ci = r * NBUF + b
            gather_desc(b, ci).wait()
            scale_buf(b)
            scatter_desc(b, ci).start()
        for b in range(NBUF):
            ci = r * NBUF + b

            @pl.when(r < N_ROUNDS - 1)
            def _():
                scatter_desc(b, ci).wait()
                gather_desc(b, ci + NBUF).start()

        return 0

    lax.fori_loop(0, N_ROUNDS, round_body, 0)

    for b in range(NBUF):
        scatter_desc(b, (N_ROUNDS - 1) * NBUF + b).wait()


@jax.jit
def kernel(input, table):
    idx = input.reshape(-1).astype(jnp.int32)
    mesh = plsc.VectorSubcoreMesh(core_axis_name="c", subcore_axis_name="s")
    out = pl.kernel(
        _emb_kernel,
        out_type=jax.ShapeDtypeStruct((B_TOTAL, D_MODEL), jnp.float32),
        mesh=mesh,
        scratch_types=[
            pltpu.VMEM((B_PER_W,), jnp.int32),
            pltpu.VMEM((CH, D_MODEL), jnp.float32),
            pltpu.VMEM((CH, D_MODEL), jnp.float32),
            pltpu.SemaphoreType.DMA,
            pltpu.SemaphoreType.DMA,
            pltpu.SemaphoreType.DMA,
            pltpu.SemaphoreType.DMA,
        ],
    )(idx, table)
    return out.reshape(input.shape + (D_MODEL,))
